# baseline (speedup 1.0000x reference)
import functools

import jax
import jax.numpy as jnp
from jax import lax
from jax.experimental import pallas as pl
from jax.experimental.pallas import tpu as pltpu
from jax.experimental.pallas import tpu_sc as plsc


def _make_sc_broadcast(B: int, S: int, D: int, dtype):
    info = plsc.get_sparse_core_info()
    NC, NS = info.num_cores, info.num_subcores
    NW = NC * NS
    assert S % NW == 0
    rows_per_w = S // NW
    chunk = 64
    n_chunks = rows_per_w // chunk

    mesh = plsc.VectorSubcoreMesh(core_axis_name="c", subcore_axis_name="s")

    @functools.partial(
        pl.kernel,
        mesh=mesh,
        out_type=jax.ShapeDtypeStruct((B, S, D), dtype),
        scratch_types=[
            pltpu.VMEM_SHARED((NS * 64, D), dtype),
            pltpu.SemaphoreType.DMA,
        ],
    )
    def broadcast_rows(table_hbm, out_hbm, shared, wsem):
        sub = lax.axis_index("s")
        wid = sub * NC + lax.axis_index("c")
        base = wid * rows_per_w
        myslice = shared.at[pl.ds(sub * 64, chunk), :]
        for j in range(n_chunks):
            r0 = base + j * chunk
            pltpu.sync_copy(table_hbm.at[pl.ds(r0, chunk), :], myslice)
            cps = []
            for b in range(B):
                cp = pltpu.make_async_copy(
                    myslice, out_hbm.at[b, pl.ds(r0, chunk), :], wsem)
                cp.start()
                cps.append(cp)
            for cp in cps:
                cp.wait()

    return broadcast_rows


def kernel(x, position_embedding):
    B, S, _ = x.shape
    _, D = position_embedding.shape
    fn = _make_sc_broadcast(B, S, D, position_embedding.dtype)
    return fn(position_embedding)


# speedup vs baseline: 1.2045x; 1.2045x over previous
import functools

import jax
import jax.numpy as jnp
from jax import lax
from jax.experimental import pallas as pl
from jax.experimental.pallas import tpu as pltpu
from jax.experimental.pallas import tpu_sc as plsc

_SPMEM_ROWS = 24


def _make_sc_broadcast(B: int, S: int, D: int, dtype):
    info = plsc.get_sparse_core_info()
    NC, NS = info.num_cores, info.num_subcores
    NW = NC * NS
    assert S % NW == 0
    rows_per_w = S // NW
    sp_rows = _SPMEM_ROWS
    st_rows = rows_per_w - sp_rows
    st_chunk = min(st_rows, 32)
    n_full, rem = divmod(st_rows, st_chunk)
    st_chunks = [st_chunk] * n_full + ([rem] if rem else [])
    st_offs = [i * st_chunk for i in range(len(st_chunks))]

    mesh = plsc.VectorSubcoreMesh(core_axis_name="c", subcore_axis_name="s")

    @functools.partial(
        pl.kernel,
        mesh=mesh,
        out_type=jax.ShapeDtypeStruct((B, S, D), dtype),
        scratch_types=[
            pltpu.VMEM((st_chunk, D), dtype),
            pltpu.VMEM_SHARED((NS * sp_rows, D), dtype),
            pltpu.SemaphoreType.DMA,
            pltpu.SemaphoreType.DMA,
            pltpu.SemaphoreType.DMA,
            pltpu.SemaphoreType.DMA,
        ],
    )
    def broadcast_rows(table_hbm, out_hbm, buf, shared, rsa, rsb, wsa, wsb):
        sub = lax.axis_index("s")
        wid = sub * NC + lax.axis_index("c")
        base = wid * rows_per_w
        sp_base = base + st_rows
        myslice = shared.at[pl.ds(sub * sp_rows, sp_rows), :]

        c0 = st_chunks[0]
        src0 = buf if c0 == st_chunk else buf.at[pl.ds(0, c0), :]
        rd_a = pltpu.make_async_copy(
            table_hbm.at[pl.ds(base + st_offs[0], c0), :], src0, rsa)
        rd_a.start()
        rd_b = pltpu.make_async_copy(
            table_hbm.at[pl.ds(sp_base, sp_rows), :], myslice, rsb)
        rd_b.start()

        rd_b.wait()
        cps_b = []
        for b in range(B):
            cp = pltpu.make_async_copy(
                myslice, out_hbm.at[b, pl.ds(sp_base, sp_rows), :], wsb)
            cp.start()
            cps_b.append(cp)

        for j, c in enumerate(st_chunks):
            r0 = base + st_offs[j]
            src = buf if c == st_chunk else buf.at[pl.ds(0, c), :]
            rd_a.wait()
            cps_a = []
            for b in range(B):
                cp = pltpu.make_async_copy(
                    src, out_hbm.at[b, pl.ds(r0, c), :], wsa)
                cp.start()
                cps_a.append(cp)
            for cp in cps_a:
                cp.wait()
            if j + 1 < len(st_chunks):
                cn = st_chunks[j + 1]
                srcn = buf if cn == st_chunk else buf.at[pl.ds(0, cn), :]
                rd_a = pltpu.make_async_copy(
                    table_hbm.at[pl.ds(base + st_offs[j + 1], cn), :],
                    srcn, rsa)
                rd_a.start()

        for cp in cps_b:
            cp.wait()

    return broadcast_rows


def kernel(x, position_embedding):
    B, S, _ = x.shape
    _, D = position_embedding.shape
    fn = _make_sc_broadcast(B, S, D, position_embedding.dtype)
    return fn(position_embedding)


# speedup vs baseline: 1.2258x; 1.0177x over previous
import functools

import jax
import jax.numpy as jnp
from jax import lax
from jax.experimental import pallas as pl
from jax.experimental.pallas import tpu as pltpu
from jax.experimental.pallas import tpu_sc as plsc


def _make_sc_broadcast(B: int, S: int, D: int, dtype):
    info = plsc.get_sparse_core_info()
    NC, NS = info.num_cores, info.num_subcores
    NW = NC * NS
    assert S % NW == 0
    rows_per_w = S // NW
    chunk = min(rows_per_w, max(8, (131071 // D) & ~7))
    n_full, rem = divmod(rows_per_w, chunk)
    chunks = [chunk] * n_full + ([rem] if rem else [])
    offs = [i * chunk for i in range(len(chunks))]

    mesh = plsc.VectorSubcoreMesh(core_axis_name="c", subcore_axis_name="s")

    @functools.partial(
        pl.kernel,
        mesh=mesh,
        out_type=jax.ShapeDtypeStruct((B, S, D), dtype),
        scratch_types=[
            pltpu.VMEM((chunk, D), dtype),
            pltpu.SemaphoreType.DMA,
        ],
    )
    def broadcast_rows(table_hbm, out_hbm, buf, wsem):
        wid = lax.axis_index("s") * NC + lax.axis_index("c")
        base = wid * rows_per_w
        for j, c in enumerate(chunks):
            r0 = base + offs[j]
            src = buf if c == chunk else buf.at[pl.ds(0, c), :]
            pltpu.sync_copy(table_hbm.at[pl.ds(r0, c), :], src)
            cps = []
            for i in range(B):
                b = (i + wid) % B
                cp = pltpu.make_async_copy(
                    src, out_hbm.at[b, pl.ds(r0, c), :], wsem)
                cp.start()
                cps.append(cp)
            for cp in cps:
                cp.wait()

    return broadcast_rows


def kernel(x, position_embedding):
    B, S, _ = x.shape
    _, D = position_embedding.shape
    fn = _make_sc_broadcast(B, S, D, position_embedding.dtype)
    return fn(position_embedding)


# speedup vs baseline: 1.2290x; 1.0026x over previous
import functools

import jax
import jax.numpy as jnp
from jax import lax
from jax.experimental import pallas as pl
from jax.experimental.pallas import tpu as pltpu
from jax.experimental.pallas import tpu_sc as plsc


def _make_sc_broadcast(B: int, S: int, D: int, dtype):
    info = plsc.get_sparse_core_info()
    NC, NS = info.num_cores, info.num_subcores
    NW = NC * NS
    assert S % NW == 0
    rows_per_w = S // NW
    chunk = min(rows_per_w, max(8, (131071 // D) & ~7))
    n_full, rem = divmod(rows_per_w, chunk)
    chunks = [chunk] * n_full + ([rem] if rem else [])
    offs = [i * chunk for i in range(len(chunks))]

    mesh = plsc.VectorSubcoreMesh(core_axis_name="c", subcore_axis_name="s")

    @functools.partial(
        pl.kernel,
        mesh=mesh,
        out_type=jax.ShapeDtypeStruct((B, S, D), dtype),
        scratch_types=[
            pltpu.VMEM((chunk, D), dtype),
            pltpu.SemaphoreType.DMA,
        ],
    )
    def broadcast_rows(table_hbm, out_hbm, buf, wsem):
        wid = lax.axis_index("s") * NC + lax.axis_index("c")
        base = wid * rows_per_w
        for j, c in enumerate(chunks):
            r0 = base + offs[j]
            src = buf if c == chunk else buf.at[pl.ds(0, c), :]
            pltpu.sync_copy(table_hbm.at[pl.ds(r0, c), :], src)
            cps = []
            for b in range(B):
                cp = pltpu.make_async_copy(
                    src, out_hbm.at[b, pl.ds(r0, c), :], wsem)
                cp.start()
                cps.append(cp)
            for cp in cps:
                cp.wait()

    return broadcast_rows


def kernel(x, position_embedding):
    B, S, _ = x.shape
    _, D = position_embedding.shape
    fn = _make_sc_broadcast(B, S, D, position_embedding.dtype)
    return fn(position_embedding)
